# baseline (speedup 1.0000x reference)
import functools

import jax
import jax.numpy as jnp
from jax import lax
from jax.experimental import pallas as pl
from jax.experimental.pallas import tpu as pltpu
from jax.experimental.pallas import tpu_sc as plsc

NUM_CLASS = 19
NIMG, H, W = 16, 512, 512
NC, NS, L = 2, 16, 16
NW = NC * NS
ROWS_W = H // 2
RCHUNK = 32
NCHUNK = ROWS_W // RCHUNK
NSLOT = 2
VECS = RCHUNK * W // L
HBINS = 368
HSZ = HBINS * L

_mesh = plsc.VectorSubcoreMesh(core_axis_name="c", subcore_axis_name="s")


@functools.partial(
    pl.kernel,
    mesh=_mesh,
    compiler_params=pltpu.CompilerParams(
        needs_layout_passes=False, use_tc_tiling_on_sc=True),
    out_type=jax.ShapeDtypeStruct((NW, HSZ), jnp.int32),
    scratch_types=[
        pltpu.VMEM((NSLOT, RCHUNK, W), jnp.int32),
        pltpu.VMEM((NSLOT, RCHUNK, W), jnp.int32),
        pltpu.VMEM((HSZ,), jnp.int32),
        pltpu.SemaphoreType.DMA,
        pltpu.SemaphoreType.DMA,
    ],
)
def _hist_kernel(pred_hbm, label_hbm, out_hbm, pbuf, lbuf, hist,
                 sem0, sem1):
    c = lax.axis_index("c")
    s = lax.axis_index("s")
    wid = c * NS + s
    img = wid // 2
    row0 = (wid % 2) * ROWS_W

    zeros = jnp.zeros((L,), jnp.int32)

    @plsc.parallel_loop(0, HBINS, unroll=8)
    def _zero(i):
        hist[pl.ds(i * L, L)] = zeros

    lane = lax.iota(jnp.int32, L)
    ones = jnp.ones((L,), jnp.int32)
    sems = (sem0, sem1)

    def issue(g, slot):
        r = row0 + g * RCHUNK
        pltpu.async_copy(
            pred_hbm.at[img, pl.ds(r, RCHUNK), :], pbuf.at[slot], sems[slot])
        pltpu.async_copy(
            label_hbm.at[img, pl.ds(r, RCHUNK), :], lbuf.at[slot], sems[slot])

    issue(0, 0)
    issue(1, 1)

    @pl.loop(0, NCHUNK // NSLOT)
    def _outer(j):
        g = j * NSLOT
        for b in range(NSLOT):
            pltpu.make_async_copy(
                pred_hbm.at[img, pl.ds(row0, RCHUNK), :], pbuf.at[b],
                sems[b]).wait()
            pltpu.make_async_copy(
                label_hbm.at[img, pl.ds(row0, RCHUNK), :], lbuf.at[b],
                sems[b]).wait()

            @plsc.parallel_loop(0, VECS, unroll=8)
            def _vbody(i):
                row = i >> 5
                col = (i & 31) * L
                pv = pbuf[b, row, pl.ds(col, L)]
                lv = lbuf[b, row, pl.ds(col, L)]
                flat = (lv * NUM_CLASS + pv) * L + lane
                plsc.addupdate_scatter(hist, [flat], ones)

            nxt = g + b + NSLOT

            @pl.when(nxt < NCHUNK)
            def _():
                issue(nxt, b)

    pltpu.sync_copy(hist, out_hbm.at[wid])


def kernel(imgPredict, imgLabel):
    parts = _hist_kernel(imgPredict, imgLabel)
    counts = parts.reshape(NW, HBINS, L).sum(axis=(0, 2))[: NUM_CLASS ** 2]
    cm = counts.reshape(NUM_CLASS, NUM_CLASS).astype(jnp.float32)

    eye = jnp.eye(NUM_CLASS, dtype=jnp.float32)
    diag = (cm * eye).sum(axis=1)
    rs = cm.sum(axis=1)
    cs = cm.sum(axis=0)
    tot = rs.sum()
    pa = diag.sum() / tot
    cpa = diag / rs
    rs_ok = (rs > 0).astype(jnp.float32)
    mpa = jnp.where(rs > 0, cpa, 0.0).sum() / rs_ok.sum()
    union = rs + cs - diag
    ciou = diag / union
    u_ok = (union > 0).astype(jnp.float32)
    miou = jnp.where(union > 0, ciou, 0.0).sum() / u_ok.sum()
    return (pa, cpa, mpa, ciou, miou)


# speedup vs baseline: 1.0273x; 1.0273x over previous
import functools

import jax
import jax.numpy as jnp
from jax import lax
from jax.experimental import pallas as pl
from jax.experimental.pallas import tpu as pltpu
from jax.experimental.pallas import tpu_sc as plsc

NUM_CLASS = 19
NIMG, H, W = 16, 512, 512
NC, NS, L = 2, 16, 16
NW = NC * NS
ROWS_W = H // 2
RCHUNK = 32
NCHUNK = ROWS_W // RCHUNK
NSLOT = 2
VECS = RCHUNK * W // L
HBINS = 368
HSZ = HBINS * L

_mesh = plsc.VectorSubcoreMesh(core_axis_name="c", subcore_axis_name="s")


@functools.partial(
    pl.kernel,
    mesh=_mesh,
    compiler_params=pltpu.CompilerParams(
        needs_layout_passes=False, use_tc_tiling_on_sc=True),
    out_type=jax.ShapeDtypeStruct((NW, HSZ), jnp.int32),
    scratch_types=[
        pltpu.VMEM((NSLOT, RCHUNK, W), jnp.int32),
        pltpu.VMEM((NSLOT, RCHUNK, W), jnp.int32),
        pltpu.VMEM((HSZ,), jnp.int32),
        pltpu.SemaphoreType.DMA,
        pltpu.SemaphoreType.DMA,
    ],
)
def _hist_kernel(pred_hbm, label_hbm, out_hbm, pbuf, lbuf, hist,
                 sem0, sem1):
    c = lax.axis_index("c")
    s = lax.axis_index("s")
    wid = c * NS + s
    img = wid // 2
    row0 = (wid % 2) * ROWS_W

    sems = (sem0, sem1)

    def issue(g, slot):
        r = row0 + g * RCHUNK
        pltpu.async_copy(
            pred_hbm.at[img, pl.ds(r, RCHUNK), :], pbuf.at[slot], sems[slot])
        pltpu.async_copy(
            label_hbm.at[img, pl.ds(r, RCHUNK), :], lbuf.at[slot], sems[slot])

    issue(0, 0)
    issue(1, 1)

    zeros = jnp.zeros((L,), jnp.int32)

    @plsc.parallel_loop(0, HBINS, unroll=8)
    def _zero(i):
        hist[pl.ds(i * L, L)] = zeros

    lane = lax.iota(jnp.int32, L)
    ones = jnp.ones((L,), jnp.int32)

    @pl.loop(0, NCHUNK // NSLOT)
    def _outer(j):
        g = j * NSLOT
        for b in range(NSLOT):
            pltpu.make_async_copy(
                pred_hbm.at[img, pl.ds(row0, RCHUNK), :], pbuf.at[b],
                sems[b]).wait()
            pltpu.make_async_copy(
                label_hbm.at[img, pl.ds(row0, RCHUNK), :], lbuf.at[b],
                sems[b]).wait()

            @plsc.parallel_loop(0, VECS, unroll=8)
            def _vbody(i):
                row = i >> 5
                col = (i & 31) * L
                pv = pbuf[b, row, pl.ds(col, L)]
                lv = lbuf[b, row, pl.ds(col, L)]
                flat = (lv * NUM_CLASS + pv) * L + lane
                plsc.addupdate_scatter(hist, [flat], ones)

            nxt = g + b + NSLOT

            @pl.when(nxt < NCHUNK)
            def _():
                issue(nxt, b)

    pltpu.sync_copy(hist, out_hbm.at[wid])


def kernel(imgPredict, imgLabel):
    parts = _hist_kernel(imgPredict, imgLabel)
    counts = parts.reshape(NW, HBINS, L).sum(axis=(0, 2))[: NUM_CLASS ** 2]
    cm = counts.reshape(NUM_CLASS, NUM_CLASS).astype(jnp.float32)

    eye = jnp.eye(NUM_CLASS, dtype=jnp.float32)
    diag = (cm * eye).sum(axis=1)
    pa = diag.sum() / cm.sum()
    cpa = diag / cm.sum(axis=1)
    mpa = jnp.nanmean(cpa)
    union = cm.sum(axis=1) + cm.sum(axis=0) - diag
    ciou = diag / union
    miou = jnp.nanmean(ciou)
    return (pa, cpa, mpa, ciou, miou)
